# Initial kernel scaffold; baseline (speedup 1.0000x reference)
import jax
import jax.numpy as jnp
from jax.experimental import pallas as pl


def kernel(x1, edge_index, W1, b1, W2, b2, W3, b3, Wl1, bl1, Wl2, bl2):
    raise NotImplementedError("write your pallas kernel here")


# speedup vs baseline: 6.4143x; 6.4143x over previous
import jax
import jax.numpy as jnp
from jax.experimental import pallas as pl

N = 10
F = 4
TB = 4096


def _chain_kernel(x_ref, m1_ref, m2_ref, m3_ref, b1_ref, b2_ref, b3_ref,
                  wl1_ref, bl1_ref, wl2_ref, bl2_ref, out_ref):
    x = x_ref[...]
    h1 = jnp.maximum(
        jnp.dot(x, m1_ref[...], preferred_element_type=jnp.float32)
        + b1_ref[...], 0.0)
    h2 = jnp.maximum(
        jnp.dot(h1, m2_ref[...], preferred_element_type=jnp.float32)
        + b2_ref[...], 0.0)
    h3 = jnp.maximum(
        jnp.dot(h2, m3_ref[...], preferred_element_type=jnp.float32)
        + b3_ref[...] + h1, 0.0)
    z = jnp.maximum(
        jnp.dot(h3, wl1_ref[...], preferred_element_type=jnp.float32)
        + bl1_ref[...], 0.0)
    out_ref[...] = (jnp.dot(z, wl2_ref[...], preferred_element_type=jnp.float32)
                    + bl2_ref[...])


def kernel(x1, edge_index, W1, b1, W2, b2, W3, b3, Wl1, bl1, Wl2, bl2):
    B = x1.shape[0]
    ei = edge_index.astype(jnp.int32)
    loop = jnp.arange(N, dtype=jnp.int32)
    src = jnp.concatenate([ei[0], loop])
    dst = jnp.concatenate([ei[1], loop])
    deg = jnp.zeros((N,), jnp.float32).at[dst].add(1.0)
    dinv = jnp.where(deg > 0, jax.lax.rsqrt(deg), 0.0)
    norm = dinv[src] * dinv[dst]
    A = jnp.zeros((N, N), jnp.float32).at[dst, src].add(norm)
    AT = A.T

    M1 = jnp.kron(AT, W1)
    M2 = jnp.kron(AT, W2)
    M3 = jnp.kron(AT, W3)
    b1r = jnp.tile(b1, N)[None, :]
    b2r = jnp.tile(b2, N)[None, :]
    b3r = jnp.tile(b3, N)[None, :]
    bl1r = bl1[None, :]
    bl2r = bl2[None, :]

    x = x1.reshape(B, N)

    full = lambda shape: pl.BlockSpec(shape, lambda i: (0, 0))
    out = pl.pallas_call(
        _chain_kernel,
        grid=(B // TB,),
        in_specs=[
            pl.BlockSpec((TB, N), lambda i: (i, 0)),
            full((N, 4 * N)), full((4 * N, 4 * N)), full((4 * N, 4 * N)),
            full((1, 4 * N)), full((1, 4 * N)), full((1, 4 * N)),
            full((4 * N, 24)), full((1, 24)), full((24, 1)), full((1, 1)),
        ],
        out_specs=pl.BlockSpec((TB, 1), lambda i: (i, 0)),
        out_shape=jax.ShapeDtypeStruct((B, 1), jnp.float32),
    )(x, M1, M2, M3, b1r, b2r, b3r, Wl1, bl1r, Wl2, bl2r)
    return out


# speedup vs baseline: 7.3087x; 1.1394x over previous
import jax
import jax.numpy as jnp
from jax import lax
from jax.experimental import pallas as pl

N = 10
E = 30
F = 4
NF = N * F
TB = 4096


def _prep_kernel(ei_ref, w1_ref, w2_ref, w3_ref, b1_ref, b2_ref, b3_ref,
                 m1_ref, m2_ref, m3_ref, b1r_ref, b2r_ref, b3r_ref):
    f32 = jnp.float32
    ei = ei_ref[...]
    ei0 = ei[0:1, :]
    ei1 = ei[1:2, :]
    niota = lax.broadcasted_iota(jnp.int32, (N, E), 0)
    ST = (ei0 == niota).astype(f32)
    DT = (ei1 == niota).astype(f32)

    deg = jnp.sum(DT, axis=1, keepdims=True) + 1.0
    dinv = lax.rsqrt(deg)

    cdim = lambda a, b: (((a,), (b,)), ((), ()))
    dinv_src = lax.dot_general(dinv, ST, cdim(0, 0),
                               preferred_element_type=f32)
    dinv_dst = lax.dot_general(dinv, DT, cdim(0, 0),
                               preferred_element_type=f32)
    norm = dinv_src * dinv_dst

    AT = lax.dot_general(ST * norm, DT, cdim(1, 1),
                         preferred_element_type=f32)
    ii = lax.broadcasted_iota(jnp.int32, (N, N), 0)
    jj = lax.broadcasted_iota(jnp.int32, (N, N), 1)
    AT = AT + jnp.where(ii == jj, dinv * dinv, 0.0)

    mi = lax.broadcasted_iota(jnp.int32, (N, NF), 0)
    ji = lax.broadcasted_iota(jnp.int32, (N, NF), 1)
    Ecol = (ji // F == mi).astype(f32)
    fi = lax.broadcasted_iota(jnp.int32, (F, NF), 0)
    gi = lax.broadcasted_iota(jnp.int32, (F, NF), 1)
    T4 = (gi % F == fi).astype(f32)

    dot = lambda a, b: jnp.dot(a, b, preferred_element_type=f32)
    m1_ref[...] = dot(AT, Ecol) * dot(w1_ref[...], T4)
    P = lax.dot_general(Ecol, AT, cdim(0, 0),
                        preferred_element_type=f32)
    ATee = dot(P, Ecol)
    Q2 = lax.dot_general(T4, w2_ref[...], cdim(0, 0),
                         preferred_element_type=f32)
    m2_ref[...] = ATee * dot(Q2, T4)
    Q3 = lax.dot_general(T4, w3_ref[...], cdim(0, 0),
                         preferred_element_type=f32)
    m3_ref[...] = ATee * dot(Q3, T4)

    b1r_ref[...] = dot(b1_ref[...], T4)
    b2r_ref[...] = dot(b2_ref[...], T4)
    b3r_ref[...] = dot(b3_ref[...], T4)


def _chain_kernel(x_ref, m1_ref, m2_ref, m3_ref, b1_ref, b2_ref, b3_ref,
                  wl1_ref, bl1_ref, wl2_ref, bl2_ref, out_ref):
    x = x_ref[...]
    h1 = jnp.maximum(
        jnp.dot(x, m1_ref[...], preferred_element_type=jnp.float32)
        + b1_ref[...], 0.0)
    h2 = jnp.maximum(
        jnp.dot(h1, m2_ref[...], preferred_element_type=jnp.float32)
        + b2_ref[...], 0.0)
    h3 = jnp.maximum(
        jnp.dot(h2, m3_ref[...], preferred_element_type=jnp.float32)
        + b3_ref[...] + h1, 0.0)
    z = jnp.maximum(
        jnp.dot(h3, wl1_ref[...], preferred_element_type=jnp.float32)
        + bl1_ref[...], 0.0)
    out_ref[...] = (jnp.dot(z, wl2_ref[...], preferred_element_type=jnp.float32)
                    + bl2_ref[...])


def kernel(x1, edge_index, W1, b1, W2, b2, W3, b3, Wl1, bl1, Wl2, bl2):
    B = x1.shape[0]
    ei = edge_index.astype(jnp.int32)

    whole = lambda *shape: pl.BlockSpec(shape, lambda: tuple(0 for _ in shape))
    f32 = jnp.float32
    M1, M2, M3, b1r, b2r, b3r = pl.pallas_call(
        _prep_kernel,
        in_specs=[whole(2, E), whole(1, F), whole(F, F), whole(F, F),
                  whole(1, F), whole(1, F), whole(1, F)],
        out_specs=[whole(N, NF), whole(NF, NF), whole(NF, NF),
                   whole(1, NF), whole(1, NF), whole(1, NF)],
        out_shape=[jax.ShapeDtypeStruct((N, NF), f32),
                   jax.ShapeDtypeStruct((NF, NF), f32),
                   jax.ShapeDtypeStruct((NF, NF), f32),
                   jax.ShapeDtypeStruct((1, NF), f32),
                   jax.ShapeDtypeStruct((1, NF), f32),
                   jax.ShapeDtypeStruct((1, NF), f32)],
    )(ei, W1, W2, W3, b1[None, :], b2[None, :], b3[None, :])

    x = x1.reshape(B, N)
    full = lambda shape: pl.BlockSpec(shape, lambda i: (0, 0))
    out = pl.pallas_call(
        _chain_kernel,
        grid=(B // TB,),
        in_specs=[
            pl.BlockSpec((TB, N), lambda i: (i, 0)),
            full((N, NF)), full((NF, NF)), full((NF, NF)),
            full((1, NF)), full((1, NF)), full((1, NF)),
            full((NF, 24)), full((1, 24)), full((24, 1)), full((1, 1)),
        ],
        out_specs=pl.BlockSpec((TB, 1), lambda i: (i, 0)),
        out_shape=jax.ShapeDtypeStruct((B, 1), jnp.float32),
    )(x, M1, M2, M3, b1r, b2r, b3r, Wl1, bl1[None, :], Wl2, bl2[None, :])
    return out


# speedup vs baseline: 15.0732x; 2.0624x over previous
import jax
import jax.numpy as jnp
from jax import lax
from jax.experimental import pallas as pl

N = 10
E = 30
F = 4
NF = N * F
H = 24
TL = 2048


def _prep_kernel(ei_ref, w1_ref, w2_ref, w3_ref, b1_ref, b2_ref, b3_ref,
                 wl1_ref, bl1_ref, wl2_ref,
                 m1t_ref, m2t_ref, m3t_ref, b1c_ref, b2c_ref, b3c_ref,
                 wl1t_ref, bl1c_ref, wl2t_ref):
    f32 = jnp.float32
    cdim = lambda a, b: (((a,), (b,)), ((), ()))
    dg = lambda a, b, c: lax.dot_general(a, b, c, preferred_element_type=f32)
    dot = lambda a, b: jnp.dot(a, b, preferred_element_type=f32)

    ei = ei_ref[...]
    ei0 = ei[0:1, :]
    ei1 = ei[1:2, :]
    niota = lax.broadcasted_iota(jnp.int32, (N, E), 0)
    ST = (ei0 == niota).astype(f32)
    DT = (ei1 == niota).astype(f32)

    deg = jnp.sum(DT, axis=1, keepdims=True) + 1.0
    dinv = lax.rsqrt(deg)

    dinv_src = dg(dinv, ST, cdim(0, 0))
    dinv_dst = dg(dinv, DT, cdim(0, 0))
    norm = dinv_src * dinv_dst

    AT = dg(ST * norm, DT, cdim(1, 1))
    ii = lax.broadcasted_iota(jnp.int32, (N, N), 0)
    jj = lax.broadcasted_iota(jnp.int32, (N, N), 1)
    AT = AT + jnp.where(ii == jj, dinv * dinv, 0.0)

    mi = lax.broadcasted_iota(jnp.int32, (N, NF), 0)
    ji = lax.broadcasted_iota(jnp.int32, (N, NF), 1)
    Ecol = (ji // F == mi).astype(f32)
    fi = lax.broadcasted_iota(jnp.int32, (F, NF), 0)
    gi = lax.broadcasted_iota(jnp.int32, (F, NF), 1)
    T4 = (gi % F == fi).astype(f32)

    AtE = dg(Ecol, AT, cdim(0, 1))
    w1c = dg(T4, w1_ref[...], cdim(0, 1))
    m1t_ref[...] = AtE * w1c

    R = dg(Ecol, AT, cdim(0, 1))
    ATeeT = dot(R, Ecol)
    U2 = dg(T4, w2_ref[...], cdim(0, 1))
    m2t_ref[...] = ATeeT * dot(U2, T4)
    U3 = dg(T4, w3_ref[...], cdim(0, 1))
    m3t_ref[...] = ATeeT * dot(U3, T4)

    b1c_ref[...] = dg(T4, b1_ref[...], cdim(0, 1))
    b2c_ref[...] = dg(T4, b2_ref[...], cdim(0, 1))
    b3c_ref[...] = dg(T4, b3_ref[...], cdim(0, 1))

    i40a = lax.broadcasted_iota(jnp.int32, (NF, NF), 0)
    i40b = lax.broadcasted_iota(jnp.int32, (NF, NF), 1)
    I40 = (i40a == i40b).astype(f32)
    wl1t_ref[...] = dg(wl1_ref[...], I40, cdim(0, 0))
    i24a = lax.broadcasted_iota(jnp.int32, (H, H), 0)
    i24b = lax.broadcasted_iota(jnp.int32, (H, H), 1)
    I24 = (i24a == i24b).astype(f32)
    bl1c_ref[...] = dg(I24, bl1_ref[...], cdim(0, 1))
    wl2t_ref[...] = dg(wl2_ref[...], I24, cdim(0, 0))


def _chain_kernel(x_ref, m1t_ref, m2t_ref, m3t_ref, b1c_ref, b2c_ref,
                  b3c_ref, wl1t_ref, bl1c_ref, wl2t_ref, bl2_ref, out_ref):
    dot = lambda a, b: jnp.dot(a, b, preferred_element_type=jnp.float32)
    xT = x_ref[...]
    h1 = jnp.maximum(dot(m1t_ref[...], xT) + b1c_ref[...], 0.0)
    h2 = jnp.maximum(dot(m2t_ref[...], h1) + b2c_ref[...], 0.0)
    h3 = jnp.maximum(dot(m3t_ref[...], h2) + b3c_ref[...] + h1, 0.0)
    z = jnp.maximum(dot(wl1t_ref[...], h3) + bl1c_ref[...], 0.0)
    out_ref[...] = dot(wl2t_ref[...], z) + bl2_ref[...]


def kernel(x1, edge_index, W1, b1, W2, b2, W3, b3, Wl1, bl1, Wl2, bl2):
    B = x1.shape[0]
    ei = edge_index.astype(jnp.int32)

    whole = lambda *shape: pl.BlockSpec(shape, lambda: tuple(0 for _ in shape))
    f32 = jnp.float32
    sds = jax.ShapeDtypeStruct
    (M1T, M2T, M3T, b1c, b2c, b3c, Wl1T, bl1c, Wl2T) = pl.pallas_call(
        _prep_kernel,
        in_specs=[whole(2, E), whole(1, F), whole(F, F), whole(F, F),
                  whole(1, F), whole(1, F), whole(1, F),
                  whole(NF, H), whole(1, H), whole(H, 1)],
        out_specs=[whole(NF, N), whole(NF, NF), whole(NF, NF),
                   whole(NF, 1), whole(NF, 1), whole(NF, 1),
                   whole(H, NF), whole(H, 1), whole(1, H)],
        out_shape=[sds((NF, N), f32), sds((NF, NF), f32), sds((NF, NF), f32),
                   sds((NF, 1), f32), sds((NF, 1), f32), sds((NF, 1), f32),
                   sds((H, NF), f32), sds((H, 1), f32), sds((1, H), f32)],
    )(ei, W1, W2, W3, b1[None, :], b2[None, :], b3[None, :],
      Wl1, bl1[None, :], Wl2)

    xT = x1.reshape(B, N).T
    full = lambda shape: pl.BlockSpec(shape, lambda i: (0, 0))
    outT = pl.pallas_call(
        _chain_kernel,
        grid=(B // TL,),
        in_specs=[
            pl.BlockSpec((N, TL), lambda i: (0, i)),
            full((NF, N)), full((NF, NF)), full((NF, NF)),
            full((NF, 1)), full((NF, 1)), full((NF, 1)),
            full((H, NF)), full((H, 1)), full((1, H)), full((1, 1)),
        ],
        out_specs=pl.BlockSpec((1, TL), lambda i: (0, i)),
        out_shape=sds((1, B), f32),
    )(xT, M1T, M2T, M3T, b1c, b2c, b3c, Wl1T, bl1c, Wl2T, bl2[None, :])
    return outT.reshape(B, 1)


# speedup vs baseline: 21.8611x; 1.4503x over previous
import jax
import jax.numpy as jnp
from jax import lax
from jax.experimental import pallas as pl

N = 10
E = 30
F = 4
NF = N * F
H = 24
TL = 4096


def _prep_kernel(ei_ref, w1_ref, w2_ref, w3_ref, b1_ref, b2_ref, b3_ref,
                 wl1_ref, bl1_ref, wl2_ref,
                 m1t_ref, m2t_ref, m3t_ref, b1c_ref, b2c_ref, b3c_ref,
                 wl1t_ref, bl1c_ref, wl2t_ref):
    f32 = jnp.float32
    cdim = lambda a, b: (((a,), (b,)), ((), ()))
    dg = lambda a, b, c: lax.dot_general(a, b, c, preferred_element_type=f32)
    dot = lambda a, b: jnp.dot(a, b, preferred_element_type=f32)

    ei = ei_ref[...]
    ei0 = ei[0:1, :]
    ei1 = ei[1:2, :]
    niota = lax.broadcasted_iota(jnp.int32, (N, E), 0)
    ST = (ei0 == niota).astype(f32)
    DT = (ei1 == niota).astype(f32)

    deg = jnp.sum(DT, axis=1, keepdims=True) + 1.0
    dinv = lax.rsqrt(deg)

    dinv_src = dg(dinv, ST, cdim(0, 0))
    dinv_dst = dg(dinv, DT, cdim(0, 0))
    norm = dinv_src * dinv_dst

    AT = dg(ST * norm, DT, cdim(1, 1))
    ii = lax.broadcasted_iota(jnp.int32, (N, N), 0)
    jj = lax.broadcasted_iota(jnp.int32, (N, N), 1)
    AT = AT + jnp.where(ii == jj, dinv * dinv, 0.0)

    mi = lax.broadcasted_iota(jnp.int32, (N, NF), 0)
    ji = lax.broadcasted_iota(jnp.int32, (N, NF), 1)
    Ecol = (ji // F == mi).astype(f32)
    fi = lax.broadcasted_iota(jnp.int32, (F, NF), 0)
    gi = lax.broadcasted_iota(jnp.int32, (F, NF), 1)
    T4 = (gi % F == fi).astype(f32)

    AtE = dg(Ecol, AT, cdim(0, 1))
    w1c = dg(T4, w1_ref[...], cdim(0, 1))
    m1t_ref[...] = AtE * w1c

    R = dg(Ecol, AT, cdim(0, 1))
    ATeeT = dot(R, Ecol)
    U2 = dg(T4, w2_ref[...], cdim(0, 1))
    m2t_ref[...] = ATeeT * dot(U2, T4)
    U3 = dg(T4, w3_ref[...], cdim(0, 1))
    m3t_ref[...] = ATeeT * dot(U3, T4)

    b1c_ref[...] = dg(T4, b1_ref[...], cdim(0, 1))
    b2c_ref[...] = dg(T4, b2_ref[...], cdim(0, 1))
    b3c_ref[...] = dg(T4, b3_ref[...], cdim(0, 1))

    i40a = lax.broadcasted_iota(jnp.int32, (NF, NF), 0)
    i40b = lax.broadcasted_iota(jnp.int32, (NF, NF), 1)
    I40 = (i40a == i40b).astype(f32)
    wl1t_ref[...] = dg(wl1_ref[...], I40, cdim(0, 0))
    i24a = lax.broadcasted_iota(jnp.int32, (H, H), 0)
    i24b = lax.broadcasted_iota(jnp.int32, (H, H), 1)
    I24 = (i24a == i24b).astype(f32)
    bl1c_ref[...] = dg(I24, bl1_ref[...], cdim(0, 1))
    wl2t_ref[...] = dg(wl2_ref[...], I24, cdim(0, 0))


def _chain_kernel(x_ref, m1t_ref, m2t_ref, m3t_ref, b1c_ref, b2c_ref,
                  b3c_ref, wl1t_ref, bl1c_ref, wl2t_ref, bl2_ref, out_ref):
    dot = lambda a, b: jnp.dot(a, b, preferred_element_type=jnp.float32)
    xT = x_ref[...]
    h1 = jnp.maximum(dot(m1t_ref[...], xT) + b1c_ref[...], 0.0)
    h2 = jnp.maximum(dot(m2t_ref[...], h1) + b2c_ref[...], 0.0)
    h3 = jnp.maximum(dot(m3t_ref[...], h2) + b3c_ref[...] + h1, 0.0)
    z = jnp.maximum(dot(wl1t_ref[...], h3) + bl1c_ref[...], 0.0)
    out_ref[...] = dot(wl2t_ref[...], z) + bl2_ref[...]


def kernel(x1, edge_index, W1, b1, W2, b2, W3, b3, Wl1, bl1, Wl2, bl2):
    B = x1.shape[0]
    ei = edge_index.astype(jnp.int32)

    whole = lambda *shape: pl.BlockSpec(shape, lambda: tuple(0 for _ in shape))
    f32 = jnp.float32
    sds = jax.ShapeDtypeStruct
    (M1T, M2T, M3T, b1c, b2c, b3c, Wl1T, bl1c, Wl2T) = pl.pallas_call(
        _prep_kernel,
        in_specs=[whole(2, E), whole(1, F), whole(F, F), whole(F, F),
                  whole(1, F), whole(1, F), whole(1, F),
                  whole(NF, H), whole(1, H), whole(H, 1)],
        out_specs=[whole(NF, N), whole(NF, NF), whole(NF, NF),
                   whole(NF, 1), whole(NF, 1), whole(NF, 1),
                   whole(H, NF), whole(H, 1), whole(1, H)],
        out_shape=[sds((NF, N), f32), sds((NF, NF), f32), sds((NF, NF), f32),
                   sds((NF, 1), f32), sds((NF, 1), f32), sds((NF, 1), f32),
                   sds((H, NF), f32), sds((H, 1), f32), sds((1, H), f32)],
    )(ei, W1, W2, W3, b1[None, :], b2[None, :], b3[None, :],
      Wl1, bl1[None, :], Wl2)

    xT = x1.reshape(B, N).T
    full = lambda shape: pl.BlockSpec(shape, lambda i: (0, 0))
    outT = pl.pallas_call(
        _chain_kernel,
        grid=(B // TL,),
        in_specs=[
            pl.BlockSpec((N, TL), lambda i: (0, i)),
            full((NF, N)), full((NF, NF)), full((NF, NF)),
            full((NF, 1)), full((NF, 1)), full((NF, 1)),
            full((H, NF)), full((H, 1)), full((1, H)), full((1, 1)),
        ],
        out_specs=pl.BlockSpec((1, TL), lambda i: (0, i)),
        out_shape=sds((1, B), f32),
    )(xT, M1T, M2T, M3T, b1c, b2c, b3c, Wl1T, bl1c, Wl2T, bl2[None, :])
    return outT.reshape(B, 1)


# speedup vs baseline: 27.7821x; 1.2709x over previous
import jax
import jax.numpy as jnp
from jax import lax
from jax.experimental import pallas as pl

N = 10
E = 30
F = 4
NF = N * F
H = 24
TL = 8192


def _prep_kernel(ei_ref, w1_ref, w2_ref, w3_ref, b1_ref, b2_ref, b3_ref,
                 wl1_ref, bl1_ref, wl2_ref,
                 m1t_ref, m2t_ref, m3t_ref, b1c_ref, b2c_ref, b3c_ref,
                 wl1t_ref, bl1c_ref, wl2t_ref):
    f32 = jnp.float32
    cdim = lambda a, b: (((a,), (b,)), ((), ()))
    dg = lambda a, b, c: lax.dot_general(a, b, c, preferred_element_type=f32)
    dot = lambda a, b: jnp.dot(a, b, preferred_element_type=f32)

    ei = ei_ref[...]
    ei0 = ei[0:1, :]
    ei1 = ei[1:2, :]
    niota = lax.broadcasted_iota(jnp.int32, (N, E), 0)
    ST = (ei0 == niota).astype(f32)
    DT = (ei1 == niota).astype(f32)

    deg = jnp.sum(DT, axis=1, keepdims=True) + 1.0
    dinv = lax.rsqrt(deg)

    dinv_src = dg(dinv, ST, cdim(0, 0))
    dinv_dst = dg(dinv, DT, cdim(0, 0))
    norm = dinv_src * dinv_dst

    AT = dg(ST * norm, DT, cdim(1, 1))
    ii = lax.broadcasted_iota(jnp.int32, (N, N), 0)
    jj = lax.broadcasted_iota(jnp.int32, (N, N), 1)
    AT = AT + jnp.where(ii == jj, dinv * dinv, 0.0)

    mi = lax.broadcasted_iota(jnp.int32, (N, NF), 0)
    ji = lax.broadcasted_iota(jnp.int32, (N, NF), 1)
    Ecol = (ji // F == mi).astype(f32)
    fi = lax.broadcasted_iota(jnp.int32, (F, NF), 0)
    gi = lax.broadcasted_iota(jnp.int32, (F, NF), 1)
    T4 = (gi % F == fi).astype(f32)

    AtE = dg(Ecol, AT, cdim(0, 1))
    w1c = dg(T4, w1_ref[...], cdim(0, 1))
    m1t_ref[...] = AtE * w1c

    R = dg(Ecol, AT, cdim(0, 1))
    ATeeT = dot(R, Ecol)
    U2 = dg(T4, w2_ref[...], cdim(0, 1))
    m2t_ref[...] = ATeeT * dot(U2, T4)
    U3 = dg(T4, w3_ref[...], cdim(0, 1))
    m3t_ref[...] = ATeeT * dot(U3, T4)

    b1c_ref[...] = dg(T4, b1_ref[...], cdim(0, 1))
    b2c_ref[...] = dg(T4, b2_ref[...], cdim(0, 1))
    b3c_ref[...] = dg(T4, b3_ref[...], cdim(0, 1))

    i40a = lax.broadcasted_iota(jnp.int32, (NF, NF), 0)
    i40b = lax.broadcasted_iota(jnp.int32, (NF, NF), 1)
    I40 = (i40a == i40b).astype(f32)
    wl1t_ref[...] = dg(wl1_ref[...], I40, cdim(0, 0))
    i24a = lax.broadcasted_iota(jnp.int32, (H, H), 0)
    i24b = lax.broadcasted_iota(jnp.int32, (H, H), 1)
    I24 = (i24a == i24b).astype(f32)
    bl1c_ref[...] = dg(I24, bl1_ref[...], cdim(0, 1))
    wl2t_ref[...] = dg(wl2_ref[...], I24, cdim(0, 0))


def _chain_kernel(x_ref, m1t_ref, m2t_ref, m3t_ref, b1c_ref, b2c_ref,
                  b3c_ref, wl1t_ref, bl1c_ref, wl2t_ref, bl2_ref, out_ref):
    dot = lambda a, b: jnp.dot(a, b, preferred_element_type=jnp.float32)
    xT = x_ref[...]
    h1 = jnp.maximum(dot(m1t_ref[...], xT) + b1c_ref[...], 0.0)
    h2 = jnp.maximum(dot(m2t_ref[...], h1) + b2c_ref[...], 0.0)
    h3 = jnp.maximum(dot(m3t_ref[...], h2) + b3c_ref[...] + h1, 0.0)
    z = jnp.maximum(dot(wl1t_ref[...], h3) + bl1c_ref[...], 0.0)
    out_ref[...] = dot(wl2t_ref[...], z) + bl2_ref[...]


def kernel(x1, edge_index, W1, b1, W2, b2, W3, b3, Wl1, bl1, Wl2, bl2):
    B = x1.shape[0]
    ei = edge_index.astype(jnp.int32)

    whole = lambda *shape: pl.BlockSpec(shape, lambda: tuple(0 for _ in shape))
    f32 = jnp.float32
    sds = jax.ShapeDtypeStruct
    (M1T, M2T, M3T, b1c, b2c, b3c, Wl1T, bl1c, Wl2T) = pl.pallas_call(
        _prep_kernel,
        in_specs=[whole(2, E), whole(1, F), whole(F, F), whole(F, F),
                  whole(1, F), whole(1, F), whole(1, F),
                  whole(NF, H), whole(1, H), whole(H, 1)],
        out_specs=[whole(NF, N), whole(NF, NF), whole(NF, NF),
                   whole(NF, 1), whole(NF, 1), whole(NF, 1),
                   whole(H, NF), whole(H, 1), whole(1, H)],
        out_shape=[sds((NF, N), f32), sds((NF, NF), f32), sds((NF, NF), f32),
                   sds((NF, 1), f32), sds((NF, 1), f32), sds((NF, 1), f32),
                   sds((H, NF), f32), sds((H, 1), f32), sds((1, H), f32)],
    )(ei, W1, W2, W3, b1[None, :], b2[None, :], b3[None, :],
      Wl1, bl1[None, :], Wl2)

    xT = x1.reshape(B, N).T
    full = lambda shape: pl.BlockSpec(shape, lambda i: (0, 0))
    outT = pl.pallas_call(
        _chain_kernel,
        grid=(B // TL,),
        in_specs=[
            pl.BlockSpec((N, TL), lambda i: (0, i)),
            full((NF, N)), full((NF, NF)), full((NF, NF)),
            full((NF, 1)), full((NF, 1)), full((NF, 1)),
            full((H, NF)), full((H, 1)), full((1, H)), full((1, 1)),
        ],
        out_specs=pl.BlockSpec((1, TL), lambda i: (0, i)),
        out_shape=sds((1, B), f32),
    )(xT, M1T, M2T, M3T, b1c, b2c, b3c, Wl1T, bl1c, Wl2T, bl2[None, :])
    return outT.reshape(B, 1)


# speedup vs baseline: 29.7960x; 1.0725x over previous
import jax
import jax.numpy as jnp
from jax import lax
from jax.experimental import pallas as pl

N = 10
E = 30
F = 4
NF = N * F
H = 24
TL = 16384


def _prep_kernel(ei_ref, w1_ref, w2_ref, w3_ref, b1_ref, b2_ref, b3_ref,
                 wl1_ref, bl1_ref, wl2_ref,
                 m1t_ref, m2t_ref, m3t_ref, b1c_ref, b2c_ref, b3c_ref,
                 wl1t_ref, bl1c_ref, wl2t_ref):
    f32 = jnp.float32
    cdim = lambda a, b: (((a,), (b,)), ((), ()))
    dg = lambda a, b, c: lax.dot_general(a, b, c, preferred_element_type=f32)
    dot = lambda a, b: jnp.dot(a, b, preferred_element_type=f32)

    ei = ei_ref[...]
    ei0 = ei[0:1, :]
    ei1 = ei[1:2, :]
    niota = lax.broadcasted_iota(jnp.int32, (N, E), 0)
    ST = (ei0 == niota).astype(f32)
    DT = (ei1 == niota).astype(f32)

    deg = jnp.sum(DT, axis=1, keepdims=True) + 1.0
    dinv = lax.rsqrt(deg)

    dinv_src = dg(dinv, ST, cdim(0, 0))
    dinv_dst = dg(dinv, DT, cdim(0, 0))
    norm = dinv_src * dinv_dst

    AT = dg(ST * norm, DT, cdim(1, 1))
    ii = lax.broadcasted_iota(jnp.int32, (N, N), 0)
    jj = lax.broadcasted_iota(jnp.int32, (N, N), 1)
    AT = AT + jnp.where(ii == jj, dinv * dinv, 0.0)

    mi = lax.broadcasted_iota(jnp.int32, (N, NF), 0)
    ji = lax.broadcasted_iota(jnp.int32, (N, NF), 1)
    Ecol = (ji // F == mi).astype(f32)
    fi = lax.broadcasted_iota(jnp.int32, (F, NF), 0)
    gi = lax.broadcasted_iota(jnp.int32, (F, NF), 1)
    T4 = (gi % F == fi).astype(f32)

    AtE = dg(Ecol, AT, cdim(0, 1))
    w1c = dg(T4, w1_ref[...], cdim(0, 1))
    m1t_ref[...] = AtE * w1c

    R = dg(Ecol, AT, cdim(0, 1))
    ATeeT = dot(R, Ecol)
    U2 = dg(T4, w2_ref[...], cdim(0, 1))
    m2t_ref[...] = ATeeT * dot(U2, T4)
    U3 = dg(T4, w3_ref[...], cdim(0, 1))
    m3t_ref[...] = ATeeT * dot(U3, T4)

    b1c_ref[...] = dg(T4, b1_ref[...], cdim(0, 1))
    b2c_ref[...] = dg(T4, b2_ref[...], cdim(0, 1))
    b3c_ref[...] = dg(T4, b3_ref[...], cdim(0, 1))

    i40a = lax.broadcasted_iota(jnp.int32, (NF, NF), 0)
    i40b = lax.broadcasted_iota(jnp.int32, (NF, NF), 1)
    I40 = (i40a == i40b).astype(f32)
    wl1t_ref[...] = dg(wl1_ref[...], I40, cdim(0, 0))
    i24a = lax.broadcasted_iota(jnp.int32, (H, H), 0)
    i24b = lax.broadcasted_iota(jnp.int32, (H, H), 1)
    I24 = (i24a == i24b).astype(f32)
    bl1c_ref[...] = dg(I24, bl1_ref[...], cdim(0, 1))
    wl2t_ref[...] = dg(wl2_ref[...], I24, cdim(0, 0))


def _chain_kernel(x_ref, m1t_ref, m2t_ref, m3t_ref, b1c_ref, b2c_ref,
                  b3c_ref, wl1t_ref, bl1c_ref, wl2t_ref, bl2_ref, out_ref):
    dot = lambda a, b: jnp.dot(a, b, preferred_element_type=jnp.float32)
    xT = x_ref[...]
    h1 = jnp.maximum(dot(m1t_ref[...], xT) + b1c_ref[...], 0.0)
    h2 = jnp.maximum(dot(m2t_ref[...], h1) + b2c_ref[...], 0.0)
    h3 = jnp.maximum(dot(m3t_ref[...], h2) + b3c_ref[...] + h1, 0.0)
    z = jnp.maximum(dot(wl1t_ref[...], h3) + bl1c_ref[...], 0.0)
    out_ref[...] = dot(wl2t_ref[...], z) + bl2_ref[...]


def kernel(x1, edge_index, W1, b1, W2, b2, W3, b3, Wl1, bl1, Wl2, bl2):
    B = x1.shape[0]
    ei = edge_index.astype(jnp.int32)

    whole = lambda *shape: pl.BlockSpec(shape, lambda: tuple(0 for _ in shape))
    f32 = jnp.float32
    sds = jax.ShapeDtypeStruct
    (M1T, M2T, M3T, b1c, b2c, b3c, Wl1T, bl1c, Wl2T) = pl.pallas_call(
        _prep_kernel,
        in_specs=[whole(2, E), whole(1, F), whole(F, F), whole(F, F),
                  whole(1, F), whole(1, F), whole(1, F),
                  whole(NF, H), whole(1, H), whole(H, 1)],
        out_specs=[whole(NF, N), whole(NF, NF), whole(NF, NF),
                   whole(NF, 1), whole(NF, 1), whole(NF, 1),
                   whole(H, NF), whole(H, 1), whole(1, H)],
        out_shape=[sds((NF, N), f32), sds((NF, NF), f32), sds((NF, NF), f32),
                   sds((NF, 1), f32), sds((NF, 1), f32), sds((NF, 1), f32),
                   sds((H, NF), f32), sds((H, 1), f32), sds((1, H), f32)],
    )(ei, W1, W2, W3, b1[None, :], b2[None, :], b3[None, :],
      Wl1, bl1[None, :], Wl2)

    xT = x1.reshape(B, N).T
    full = lambda shape: pl.BlockSpec(shape, lambda i: (0, 0))
    outT = pl.pallas_call(
        _chain_kernel,
        grid=(B // TL,),
        in_specs=[
            pl.BlockSpec((N, TL), lambda i: (0, i)),
            full((NF, N)), full((NF, NF)), full((NF, NF)),
            full((NF, 1)), full((NF, 1)), full((NF, 1)),
            full((H, NF)), full((H, 1)), full((1, H)), full((1, 1)),
        ],
        out_specs=pl.BlockSpec((1, TL), lambda i: (0, i)),
        out_shape=sds((1, B), f32),
    )(xT, M1T, M2T, M3T, b1c, b2c, b3c, Wl1T, bl1c, Wl2T, bl2[None, :])
    return outT.reshape(B, 1)
